# baseline (speedup 1.0000x reference)
import functools

import jax
import jax.numpy as jnp
from jax import lax
from jax.experimental import pallas as pl
from jax.experimental.pallas import tpu as pltpu

CPAD = 16


def _decoder_block_kernel(x_ref, xe_ref, w0x_ref, w0ep_ref, w0e2_ref, b0_ref,
                          w1_ref, b1_ref, o_ref,
                          xpad_ref, epad_ref, mpad_ref, *, row_tile):
    Ho = o_ref.shape[1]
    Wo = o_ref.shape[2]
    Cenc = xe_ref.shape[3]
    R = row_tile
    Rl = R // 2
    n_strips = Ho // R

    @pl.when(pl.program_id(0) == 0)
    def _():
        for ref in (xpad_ref, epad_ref, mpad_ref):
            h, w, c = ref.shape
            zrow = jnp.zeros((1, w, c), ref.dtype)
            ref[0:1, :, :] = zrow
            ref[h - 1:h, :, :] = zrow
            ref[:, 0:CPAD, :] = jnp.zeros((h, CPAD, c), ref.dtype)
            ref[:, CPAD + Wo:, :] = jnp.zeros((h, w - CPAD - Wo, c), ref.dtype)

    def copy_strip(s):
        r0 = pl.multiple_of(s * R, R)
        rl = pl.multiple_of(s * Rl, Rl)
        xc = x_ref[0, pl.ds(rl, Rl), :, :]
        xu = jnp.repeat(xc, 2, axis=0)
        xpad_ref[pl.ds(r0 + 1, R), CPAD:CPAD + Wo, :] = xu
        xe_s = xe_ref[0, pl.ds(r0, R), :, :]
        epad_ref[pl.ds(r0 + 1, R), CPAD:CPAD + Wo, 0:Cenc] = xe_s
        epad_ref[pl.ds(r0, R), CPAD:CPAD + Wo, Cenc:2 * Cenc] = xe_s

    copy_strip(0)

    def conv0_body(s, carry):
        r0 = pl.multiple_of(s * R, R)

        @pl.when(s < n_strips - 1)
        def _():
            copy_strip(s + 1)

        acc = jnp.zeros((R, Wo, mpad_ref.shape[2]), jnp.float32)
        for ky in range(3):
            for kx in range(3):
                patch = xpad_ref[pl.ds(r0 + ky, R),
                                 CPAD - 1 + kx:CPAD - 1 + kx + Wo, :]
                acc = acc + lax.dot_general(
                    patch, w0x_ref[ky, kx],
                    dimension_numbers=(((2,), (0,)), ((), ())),
                    preferred_element_type=jnp.float32)
        for kx in range(3):
            patch = epad_ref[pl.ds(r0, R),
                             CPAD - 1 + kx:CPAD - 1 + kx + Wo, 0:2 * Cenc]
            acc = acc + lax.dot_general(
                patch, w0ep_ref[kx],
                dimension_numbers=(((2,), (0,)), ((), ())),
                preferred_element_type=jnp.float32)
            patch = epad_ref[pl.ds(r0 + 1, R),
                             CPAD - 1 + kx:CPAD - 1 + kx + Wo, Cenc:2 * Cenc]
            acc = acc + lax.dot_general(
                patch, w0e2_ref[kx],
                dimension_numbers=(((2,), (0,)), ((), ())),
                preferred_element_type=jnp.float32)
        y = jnp.maximum(acc + b0_ref[0], 0.0)
        mpad_ref[pl.ds(r0 + 1, R), CPAD:CPAD + Wo, :] = y.astype(mpad_ref.dtype)
        return carry

    lax.fori_loop(0, n_strips, conv0_body, 0)

    def conv_strip(r0, srcs_and_weights, bias, n_out):
        acc = jnp.zeros((R, Wo, n_out), jnp.float32)
        for src_ref, w_ref in srcs_and_weights:
            for ky in range(3):
                for kx in range(3):
                    patch = src_ref[pl.ds(r0 + ky, R),
                                    CPAD - 1 + kx:CPAD - 1 + kx + Wo, :]
                    acc = acc + lax.dot_general(
                        patch, w_ref[ky, kx],
                        dimension_numbers=(((2,), (0,)), ((), ())),
                        preferred_element_type=jnp.float32)
        return jnp.maximum(acc + bias, 0.0)

    def conv1_body(s, carry):
        r0 = pl.multiple_of(s * R, R)
        y = conv_strip(r0, ((mpad_ref, w1_ref),), b1_ref[0], o_ref.shape[3])
        o_ref[0, pl.ds(r0, R), :, :] = y.astype(o_ref.dtype)
        return carry

    lax.fori_loop(0, n_strips, conv1_body, 0)


def _fold_bn(w, b, gamma, beta, mean, var, eps):
    scale = gamma / jnp.sqrt(var + eps)
    return w * scale, (b - mean) * scale + beta


def kernel(x, x_enc, w0, b0, g0, be0, m0, v0, w1, b1, g1, be1, m1, v1,
           *, eps=1e-5, row_tile=16):
    B, H, W, C_x = x.shape
    C_enc = x_enc.shape[-1]
    Cmid = w0.shape[-1]
    Cout = w1.shape[-1]

    w0f, b0f = _fold_bn(w0, b0, g0, be0, m0, v0, eps)
    w1f, b1f = _fold_bn(w1, b1, g1, be1, m1, v1, eps)
    w0x = w0f[:, :, :C_x, :].astype(jnp.bfloat16)
    w0e = w0f[:, :, C_x:, :]
    w0ep = jnp.concatenate([w0e[0], w0e[1]], axis=1).astype(jnp.bfloat16)
    w0e2 = w0e[2].astype(jnp.bfloat16)
    w1b = w1f.astype(jnp.bfloat16)

    xb = jnp.repeat(x.astype(jnp.bfloat16), 2, axis=2)
    xeb = x_enc.astype(jnp.bfloat16)

    Ho, Wo = 2 * H, 2 * W
    assert Ho % row_tile == 0 and row_tile % 2 == 0
    pad_cols = Wo + 2 * CPAD

    conv_kernel = functools.partial(_decoder_block_kernel, row_tile=row_tile)

    return pl.pallas_call(
        conv_kernel,
        out_shape=jax.ShapeDtypeStruct((B, Ho, Wo, Cout), x.dtype),
        grid_spec=pltpu.PrefetchScalarGridSpec(
            num_scalar_prefetch=0,
            grid=(B,),
            in_specs=[
                pl.BlockSpec((1, H, Wo, C_x), lambda b: (b, 0, 0, 0)),
                pl.BlockSpec((1, Ho, Wo, C_enc), lambda b: (b, 0, 0, 0)),
                pl.BlockSpec((3, 3, C_x, Cmid), lambda b: (0, 0, 0, 0)),
                pl.BlockSpec((3, 2 * C_enc, Cmid), lambda b: (0, 0, 0)),
                pl.BlockSpec((3, C_enc, Cmid), lambda b: (0, 0, 0)),
                pl.BlockSpec((1, Cmid), lambda b: (0, 0)),
                pl.BlockSpec((3, 3, Cmid, Cout), lambda b: (0, 0, 0, 0)),
                pl.BlockSpec((1, Cout), lambda b: (0, 0)),
            ],
            out_specs=pl.BlockSpec((1, Ho, Wo, Cout), lambda b: (b, 0, 0, 0)),
            scratch_shapes=[
                pltpu.VMEM((Ho + 2, pad_cols, C_x), jnp.bfloat16),
                pltpu.VMEM((Ho + 1, pad_cols, 2 * C_enc), jnp.bfloat16),
                pltpu.VMEM((Ho + 2, pad_cols, Cmid), jnp.bfloat16),
            ],
        ),
        compiler_params=pltpu.CompilerParams(
            dimension_semantics=("arbitrary",),
            vmem_limit_bytes=64 * 1024 * 1024,
        ),
    )(xb, xeb, w0x, w0ep, w0e2, b0f.reshape(1, -1), w1b, b1f.reshape(1, -1))


# speedup vs baseline: 1.0900x; 1.0900x over previous
import functools

import jax
import jax.numpy as jnp
from jax import lax
from jax.experimental import pallas as pl
from jax.experimental.pallas import tpu as pltpu

CPAD = 16


def _decoder_block_kernel(x_ref, xe_ref, w_ref, b_ref, o_ref,
                          xpad_ref, epad_ref, mpad_ref, *, row_tile):
    Ho = o_ref.shape[1]
    Wo = o_ref.shape[2]
    Cx = x_ref.shape[3]
    Cenc = xe_ref.shape[3]
    Cmid = mpad_ref.shape[2]
    R = row_tile
    Rl = R // 2
    n_strips = Ho // R
    o_w0x = 0
    o_w0ep = 9 * Cx
    o_w0e2 = o_w0ep + 6 * Cenc
    o_w1 = o_w0e2 + 3 * Cenc

    @pl.when(pl.program_id(0) == 0)
    def _():
        for ref in (xpad_ref, epad_ref, mpad_ref):
            h, w, c = ref.shape
            zrow = jnp.zeros((1, w, c), ref.dtype)
            ref[0:1, :, :] = zrow
            ref[h - 1:h, :, :] = zrow
            ref[:, 0:CPAD, :] = jnp.zeros((h, CPAD, c), ref.dtype)
            ref[:, CPAD + Wo:, :] = jnp.zeros((h, w - CPAD - Wo, c), ref.dtype)

    def copy_strip(s):
        r0 = pl.multiple_of(s * R, R)
        rl = pl.multiple_of(s * Rl, Rl)
        xl = x_ref[0, pl.ds(rl, Rl), :, :]
        xc = jnp.repeat(xl, 2, axis=1)
        xu = jnp.repeat(xc, 2, axis=0)
        xpad_ref[pl.ds(r0 + 1, R), CPAD:CPAD + Wo, :] = xu
        xe_s = xe_ref[0, pl.ds(r0, R), :, :]
        epad_ref[pl.ds(r0 + 1, R), CPAD:CPAD + Wo, 0:Cenc] = xe_s
        epad_ref[pl.ds(r0, R), CPAD:CPAD + Wo, Cenc:2 * Cenc] = xe_s

    copy_strip(0)

    def conv0_body(s, carry):
        r0 = pl.multiple_of(s * R, R)

        @pl.when(s < n_strips - 1)
        def _():
            copy_strip(s + 1)

        acc = jnp.zeros((R, Wo, Cmid), jnp.float32)
        for ky in range(3):
            for kx in range(3):
                patch = xpad_ref[pl.ds(r0 + ky, R),
                                 CPAD - 1 + kx:CPAD - 1 + kx + Wo, :]
                w = w_ref[o_w0x + (ky * 3 + kx) * Cx:
                          o_w0x + (ky * 3 + kx + 1) * Cx, :]
                acc = acc + lax.dot_general(
                    patch, w,
                    dimension_numbers=(((2,), (0,)), ((), ())),
                    preferred_element_type=jnp.float32)
        for kx in range(3):
            patch = epad_ref[pl.ds(r0, R),
                             CPAD - 1 + kx:CPAD - 1 + kx + Wo, 0:2 * Cenc]
            w = w_ref[o_w0ep + kx * 2 * Cenc:o_w0ep + (kx + 1) * 2 * Cenc, :]
            acc = acc + lax.dot_general(
                patch, w,
                dimension_numbers=(((2,), (0,)), ((), ())),
                preferred_element_type=jnp.float32)
            patch = epad_ref[pl.ds(r0 + 1, R),
                             CPAD - 1 + kx:CPAD - 1 + kx + Wo, Cenc:2 * Cenc]
            w = w_ref[o_w0e2 + kx * Cenc:o_w0e2 + (kx + 1) * Cenc, :]
            acc = acc + lax.dot_general(
                patch, w,
                dimension_numbers=(((2,), (0,)), ((), ())),
                preferred_element_type=jnp.float32)
        y = jnp.maximum(acc + b_ref[0, 0:Cmid], 0.0)
        mpad_ref[pl.ds(r0 + 1, R), CPAD:CPAD + Wo, :] = y.astype(mpad_ref.dtype)
        return carry

    lax.fori_loop(0, n_strips, conv0_body, 0)

    def conv1_body(s, carry):
        r0 = pl.multiple_of(s * R, R)
        acc = jnp.zeros((R, Wo, o_ref.shape[3]), jnp.float32)
        for ky in range(3):
            for kx in range(3):
                patch = mpad_ref[pl.ds(r0 + ky, R),
                                 CPAD - 1 + kx:CPAD - 1 + kx + Wo, :]
                w = w_ref[o_w1 + (ky * 3 + kx) * Cmid:
                          o_w1 + (ky * 3 + kx + 1) * Cmid, :]
                acc = acc + lax.dot_general(
                    patch, w,
                    dimension_numbers=(((2,), (0,)), ((), ())),
                    preferred_element_type=jnp.float32)
        y = jnp.maximum(acc + b_ref[1, 0:o_ref.shape[3]], 0.0)
        o_ref[0, pl.ds(r0, R), :, :] = y.astype(o_ref.dtype)
        return carry

    lax.fori_loop(0, n_strips, conv1_body, 0)


def _fold_bn(w, b, gamma, beta, mean, var, eps):
    scale = gamma / jnp.sqrt(var + eps)
    return w * scale, (b - mean) * scale + beta


def kernel(x, x_enc, w0, b0, g0, be0, m0, v0, w1, b1, g1, be1, m1, v1,
           *, eps=1e-5, row_tile=16):
    B, H, W, C_x = x.shape
    C_enc = x_enc.shape[-1]
    Cmid = w0.shape[-1]
    Cout = w1.shape[-1]

    w0f, b0f = _fold_bn(w0, b0, g0, be0, m0, v0, eps)
    w1f, b1f = _fold_bn(w1, b1, g1, be1, m1, v1, eps)
    w0x = w0f[:, :, :C_x, :].reshape(9 * C_x, Cmid)
    w0e = w0f[:, :, C_x:, :]
    w0ep = jnp.concatenate([w0e[0], w0e[1]], axis=1).reshape(6 * C_enc, Cmid)
    w0e2 = w0e[2].reshape(3 * C_enc, Cmid)
    w1r = w1f.reshape(9 * Cmid, Cout)
    wpack = jnp.concatenate([w0x, w0ep, w0e2, w1r], axis=0).astype(jnp.bfloat16)
    nb = max(Cmid, Cout)
    bpack = jnp.stack([jnp.pad(b0f, (0, nb - Cmid)),
                       jnp.pad(b1f, (0, nb - Cout))], axis=0)

    xb = x.astype(jnp.bfloat16)
    xeb = x_enc.astype(jnp.bfloat16)

    Ho, Wo = 2 * H, 2 * W
    assert Ho % row_tile == 0 and row_tile % 2 == 0
    pad_cols = Wo + 2 * CPAD

    conv_kernel = functools.partial(_decoder_block_kernel, row_tile=row_tile)

    return pl.pallas_call(
        conv_kernel,
        out_shape=jax.ShapeDtypeStruct((B, Ho, Wo, Cout), x.dtype),
        grid_spec=pltpu.PrefetchScalarGridSpec(
            num_scalar_prefetch=0,
            grid=(B,),
            in_specs=[
                pl.BlockSpec((1, H, W, C_x), lambda b: (b, 0, 0, 0)),
                pl.BlockSpec((1, Ho, Wo, C_enc), lambda b: (b, 0, 0, 0)),
                pl.BlockSpec(wpack.shape, lambda b: (0, 0)),
                pl.BlockSpec(bpack.shape, lambda b: (0, 0)),
            ],
            out_specs=pl.BlockSpec((1, Ho, Wo, Cout), lambda b: (b, 0, 0, 0)),
            scratch_shapes=[
                pltpu.VMEM((Ho + 2, pad_cols, C_x), jnp.bfloat16),
                pltpu.VMEM((Ho + 1, pad_cols, 2 * C_enc), jnp.bfloat16),
                pltpu.VMEM((Ho + 2, pad_cols, Cmid), jnp.bfloat16),
            ],
        ),
        compiler_params=pltpu.CompilerParams(
            dimension_semantics=("arbitrary",),
            vmem_limit_bytes=64 * 1024 * 1024,
        ),
    )(xb, xeb, wpack, bpack)
